# Initial kernel scaffold; baseline (speedup 1.0000x reference)
import jax
import jax.numpy as jnp
from jax.experimental import pallas as pl


def kernel(x_q, x_kv, Wq, bq, Wk, bk, Wv, bv, Wo, bo):
    raise NotImplementedError("write your pallas kernel here")


# speedup vs baseline: 18.4035x; 18.4035x over previous
import math

import numpy as np
import jax
import jax.numpy as jnp
from jax import lax
from jax.experimental import pallas as pl
from jax.experimental.pallas import tpu as pltpu

_B = 4
_L = 2048
_DM = 1024
_H = 16
_DH = 64
_BH = _B * _H
_K = max(1, int(1.0 * math.log(_L + 1)))
_KP = 8
_F = _L // 2 + 1
_FP = 1152
_FB = 128
_HW = _DM // _DH
_LB = 512


def _build_basis():
    f = np.arange(_FP, dtype=np.float64)[:, None]
    t = np.arange(_L, dtype=np.float64)[None, :]
    ang = 2.0 * np.pi * f * t / _L
    valid = (f < _F).astype(np.float64)
    c = np.cos(ang) * valid
    s = np.sin(ang) * valid
    w = np.where((f[:, 0] == 0) | (f[:, 0] == _L // 2), 1.0, 2.0)
    w = (w * valid[:, 0]) / (_L * _DH)
    c2 = w[:, None] * c
    s2 = w[:, None] * s
    return (c.astype(np.float32), s.astype(np.float32),
            c2.astype(np.float32), s2.astype(np.float32))


_C_NP, _S_NP, _C2_NP, _S2_NP = _build_basis()
_SEG_NP = (np.arange(_DM)[:, None] // _DH ==
           np.arange(_HW)[None, :]).astype(np.float32)


def _dot(a, b, dims, precision=lax.Precision.HIGHEST):
    return lax.dot_general(a, b, dimension_numbers=(dims, ((), ())),
                           preferred_element_type=jnp.float32,
                           precision=precision)


def _proj_kernel(xq_ref, xkv_ref, wq_ref, wk_ref, wv_ref,
                 bq_ref, bk_ref, bv_ref, q_ref, k_ref, v_ref):
    xq = xq_ref[0]
    xkv = xkv_ref[0]
    p = lax.Precision.DEFAULT
    q_ref[0] = _dot(xq, wq_ref[...], ((1,), (1,)), p) + bq_ref[...]
    k_ref[0] = _dot(xkv, wk_ref[...], ((1,), (1,)), p) + bk_ref[...]
    v_ref[0] = _dot(xkv, wv_ref[...], ((1,), (1,)), p) + bv_ref[...]


def _projections(x_q, x_kv, Wq, bq, Wk, bk, Wv, bv):
    nl = _L // _LB
    grid = (_B, nl)
    w_spec = pl.BlockSpec((_DM, _DM), lambda b, l: (0, 0))
    b_spec = pl.BlockSpec((1, _DM), lambda b, l: (0, 0))
    x_spec = pl.BlockSpec((1, _LB, _DM), lambda b, l: (b, l, 0))
    return pl.pallas_call(
        _proj_kernel,
        grid=grid,
        in_specs=[x_spec, x_spec, w_spec, w_spec, w_spec,
                  b_spec, b_spec, b_spec],
        out_specs=[x_spec, x_spec, x_spec],
        out_shape=[
            jax.ShapeDtypeStruct((_B, _L, _DM), jnp.float32),
            jax.ShapeDtypeStruct((_B, _L, _DM), jnp.float32),
            jax.ShapeDtypeStruct((_B, _L, _DM), jnp.float32),
        ],
    )(x_q, x_kv, Wq, Wk, Wv, bq.reshape(1, _DM), bk.reshape(1, _DM),
      bv.reshape(1, _DM))


def _dft_kernel(c_ref, s_ref, qt_ref, kt_ref, seg_ref, pre_ref, pim_ref):
    cm = c_ref[...]
    sm = s_ref[...]
    qt = qt_ref[0]
    kt = kt_ref[0]
    qfr = _dot(cm, qt, ((1,), (0,)))
    qfs = _dot(sm, qt, ((1,), (0,)))
    kfr = _dot(cm, kt, ((1,), (0,)))
    kfs = _dot(sm, kt, ((1,), (0,)))
    pre = qfr * kfr + qfs * kfs
    pim = qfr * kfs - qfs * kfr
    seg = seg_ref[...]
    pre_ref[0] = _dot(pre, seg, ((1,), (0,)))
    pim_ref[0] = _dot(pim, seg, ((1,), (0,)))


def _cross_spectrum(q, k, cmat, smat, seg):
    nfp = _FP // _FB
    grid = (_B, nfp)
    basis_spec = pl.BlockSpec((_FB, _L), lambda c, i: (i, 0))
    col_spec = pl.BlockSpec((1, _L, _DM), lambda c, i: (c, 0, 0))
    seg_spec = pl.BlockSpec((_DM, _HW), lambda c, i: (0, 0))
    p_spec = pl.BlockSpec((1, _FB, _HW), lambda c, i: (c, i, 0))
    return pl.pallas_call(
        _dft_kernel,
        grid=grid,
        in_specs=[basis_spec, basis_spec, col_spec, col_spec, seg_spec],
        out_specs=[p_spec, p_spec],
        out_shape=[
            jax.ShapeDtypeStruct((_B, _FP, _HW), jnp.float32),
            jax.ShapeDtypeStruct((_B, _FP, _HW), jnp.float32),
        ],
        compiler_params=pltpu.CompilerParams(
            vmem_limit_bytes=100 * 1024 * 1024),
    )(cmat, smat, q, k, seg)


def _corr_topk_kernel(pre_ref, pim_ref, c2_ref, s2_ref, attn_ref, delay_ref):
    corr = (_dot(pre_ref[...], c2_ref[...], ((1,), (0,))) -
            _dot(pim_ref[...], s2_ref[...], ((1,), (0,))))
    col = lax.broadcasted_iota(jnp.int32, (_BH, _L), 1)
    vals = []
    idxs = []
    cur = corr
    for _ in range(_K):
        m = jnp.max(cur, axis=1, keepdims=True)
        im = jnp.argmax(cur, axis=1, keepdims=True).astype(jnp.int32)
        vals.append(m)
        idxs.append(im)
        cur = jnp.where(col == im, -jnp.inf, cur)
    wts = jnp.concatenate(vals, axis=1)
    dly = jnp.concatenate(idxs, axis=1)
    mw = jnp.max(wts, axis=1, keepdims=True)
    e = jnp.exp(wts - mw)
    attn = e / jnp.sum(e, axis=1, keepdims=True)
    attn_ref[...] = jnp.concatenate(
        [attn, jnp.zeros((_BH, _KP - _K), jnp.float32)], axis=1)
    delay_ref[...] = jnp.concatenate(
        [dly, jnp.zeros((_BH, _KP - _K), jnp.int32)], axis=1)


def _corr_topk(pre_t, pim_t, c2mat, s2mat):
    return pl.pallas_call(
        _corr_topk_kernel,
        in_specs=[pl.BlockSpec((_BH, _FP), lambda: (0, 0)),
                  pl.BlockSpec((_BH, _FP), lambda: (0, 0)),
                  pl.BlockSpec((_FP, _L), lambda: (0, 0)),
                  pl.BlockSpec((_FP, _L), lambda: (0, 0))],
        out_specs=[pl.BlockSpec((_BH, _KP), lambda: (0, 0)),
                   pl.BlockSpec((_BH, _KP), lambda: (0, 0))],
        out_shape=[jax.ShapeDtypeStruct((_BH, _KP), jnp.float32),
                   jax.ShapeDtypeStruct((_BH, _KP), jnp.int32)],
    )(pre_t, pim_t, c2mat, s2mat)


def _agg_kernel(delay_ref, attn_ref, v2_ref, out_ref):
    h = pl.program_id(0)
    acc = attn_ref[h, 0] * v2_ref[0, pl.ds(_L - delay_ref[h, 0], _L), :]
    for j in range(1, _K):
        d = delay_ref[h, j]
        acc = acc + attn_ref[h, j] * v2_ref[0, pl.ds(_L - d, _L), :]
    out_ref[0] = acc


def _aggregate(delays, attn, v2):
    return pl.pallas_call(
        _agg_kernel,
        grid=(_BH,),
        in_specs=[pl.BlockSpec(memory_space=pltpu.SMEM),
                  pl.BlockSpec(memory_space=pltpu.SMEM),
                  pl.BlockSpec((1, 2 * _L, _DH), lambda h: (h, 0, 0))],
        out_specs=pl.BlockSpec((1, _L, _DH), lambda h: (h, 0, 0)),
        out_shape=jax.ShapeDtypeStruct((_BH, _L, _DH), jnp.float32),
    )(delays, attn, v2)


def _out_kernel(x_ref, w_ref, b_ref, y_ref):
    y_ref[...] = _dot(x_ref[...], w_ref[...], ((1,), (1,)),
                      lax.Precision.DEFAULT) + b_ref[...]


def _out_proj(ctx, Wo, bo):
    nb = (_B * _L) // _LB
    return pl.pallas_call(
        _out_kernel,
        grid=(nb,),
        in_specs=[pl.BlockSpec((_LB, _DM), lambda i: (i, 0)),
                  pl.BlockSpec((_DM, _DM), lambda i: (0, 0)),
                  pl.BlockSpec((1, _DM), lambda i: (0, 0))],
        out_specs=pl.BlockSpec((_LB, _DM), lambda i: (i, 0)),
        out_shape=jax.ShapeDtypeStruct((_B * _L, _DM), jnp.float32),
    )(ctx, Wo, bo.reshape(1, _DM))


def kernel(x_q, x_kv, Wq, bq, Wk, bk, Wv, bv, Wo, bo):
    cmat = jnp.asarray(_C_NP)
    smat = jnp.asarray(_S_NP)
    c2mat = jnp.asarray(_C2_NP)
    s2mat = jnp.asarray(_S2_NP)
    seg = jnp.asarray(_SEG_NP)

    q, k, v = _projections(x_q, x_kv, Wq, bq, Wk, bk, Wv, bv)

    pre3, pim3 = _cross_spectrum(q, k, cmat, smat, seg)
    pre_t = pre3.transpose(0, 2, 1).reshape(_BH, _FP)
    pim_t = pim3.transpose(0, 2, 1).reshape(_BH, _FP)

    attn, delays = _corr_topk(pre_t, pim_t, c2mat, s2mat)

    v4 = v.reshape(_B, _L, _H, _DH).transpose(0, 2, 1, 3).reshape(_BH, _L, _DH)
    v2 = jnp.concatenate([v4, v4], axis=1)

    agg = _aggregate(delays, attn, v2)

    ctx = agg.reshape(_B, _H, _L, _DH).transpose(0, 2, 1, 3).reshape(
        _B * _L, _DM)
    out = _out_proj(ctx, Wo, bo)
    return out.reshape(_B, _L, _DM)


# speedup vs baseline: 23.5855x; 1.2816x over previous
import math

import numpy as np
import jax
import jax.numpy as jnp
from jax import lax
from jax.experimental import pallas as pl
from jax.experimental.pallas import tpu as pltpu

_B = 4
_L = 2048
_DM = 1024
_H = 16
_DH = 64
_BH = _B * _H
_K = max(1, int(1.0 * math.log(_L + 1)))
_KP = 8
_F = _L // 2 + 1
_FP = 1152
_FB = 128
_HW = _DM // _DH
_LB = 512


def _build_basis():
    f = np.arange(_FP, dtype=np.float64)[:, None]
    t = np.arange(_L, dtype=np.float64)[None, :]
    ang = 2.0 * np.pi * f * t / _L
    valid = (f < _F).astype(np.float64)
    c = np.cos(ang) * valid
    s = np.sin(ang) * valid
    w = np.where((f[:, 0] == 0) | (f[:, 0] == _L // 2), 1.0, 2.0)
    w = (w * valid[:, 0]) / (_L * _DH)
    c2 = w[:, None] * c
    s2 = w[:, None] * s
    return (c.astype(np.float32), s.astype(np.float32),
            c2.astype(np.float32), s2.astype(np.float32))


_C_NP, _S_NP, _C2_NP, _S2_NP = _build_basis()
_SEG_NP = (np.arange(_DM)[:, None] // _DH ==
           np.arange(_HW)[None, :]).astype(np.float32)


def _dot(a, b, dims, precision=lax.Precision.HIGHEST):
    return lax.dot_general(a, b, dimension_numbers=(dims, ((), ())),
                           preferred_element_type=jnp.float32,
                           precision=precision)


def _proj_kernel(xq_ref, xkv_ref, wq_ref, wk_ref, wv_ref,
                 bq_ref, bk_ref, bv_ref, q_ref, k_ref, v_ref):
    xq = xq_ref[0]
    xkv = xkv_ref[0]
    p = lax.Precision.DEFAULT
    q_ref[0] = _dot(xq, wq_ref[...], ((1,), (1,)), p) + bq_ref[...]
    k_ref[0] = _dot(xkv, wk_ref[...], ((1,), (1,)), p) + bk_ref[...]
    v_ref[0] = _dot(xkv, wv_ref[...], ((1,), (1,)), p) + bv_ref[...]


def _projections(x_q, x_kv, Wq, bq, Wk, bk, Wv, bv):
    nl = _L // _LB
    grid = (_B, nl)
    w_spec = pl.BlockSpec((_DM, _DM), lambda b, l: (0, 0))
    b_spec = pl.BlockSpec((1, _DM), lambda b, l: (0, 0))
    x_spec = pl.BlockSpec((1, _LB, _DM), lambda b, l: (b, l, 0))
    return pl.pallas_call(
        _proj_kernel,
        grid=grid,
        in_specs=[x_spec, x_spec, w_spec, w_spec, w_spec,
                  b_spec, b_spec, b_spec],
        out_specs=[x_spec, x_spec, x_spec],
        out_shape=[
            jax.ShapeDtypeStruct((_B, _L, _DM), jnp.float32),
            jax.ShapeDtypeStruct((_B, _L, _DM), jnp.float32),
            jax.ShapeDtypeStruct((_B, _L, _DM), jnp.float32),
        ],
    )(x_q, x_kv, Wq, Wk, Wv, bq.reshape(1, _DM), bk.reshape(1, _DM),
      bv.reshape(1, _DM))


def _dft_kernel(c_ref, s_ref, qt_ref, kt_ref, seg_ref, pre_ref, pim_ref):
    cm = c_ref[...]
    sm = s_ref[...]
    qt = qt_ref[0]
    kt = kt_ref[0]
    qfr = _dot(cm, qt, ((1,), (0,)))
    qfs = _dot(sm, qt, ((1,), (0,)))
    kfr = _dot(cm, kt, ((1,), (0,)))
    kfs = _dot(sm, kt, ((1,), (0,)))
    pre = qfr * kfr + qfs * kfs
    pim = qfr * kfs - qfs * kfr
    seg = seg_ref[...]
    pre_ref[0] = _dot(pre, seg, ((1,), (0,)))
    pim_ref[0] = _dot(pim, seg, ((1,), (0,)))


def _cross_spectrum(q, k, cmat, smat, seg):
    nfp = _FP // _FB
    grid = (_B, nfp)
    basis_spec = pl.BlockSpec((_FB, _L), lambda c, i: (i, 0))
    col_spec = pl.BlockSpec((1, _L, _DM), lambda c, i: (c, 0, 0))
    seg_spec = pl.BlockSpec((_DM, _HW), lambda c, i: (0, 0))
    p_spec = pl.BlockSpec((1, _FB, _HW), lambda c, i: (c, i, 0))
    return pl.pallas_call(
        _dft_kernel,
        grid=grid,
        in_specs=[basis_spec, basis_spec, col_spec, col_spec, seg_spec],
        out_specs=[p_spec, p_spec],
        out_shape=[
            jax.ShapeDtypeStruct((_B, _FP, _HW), jnp.float32),
            jax.ShapeDtypeStruct((_B, _FP, _HW), jnp.float32),
        ],
        compiler_params=pltpu.CompilerParams(
            vmem_limit_bytes=100 * 1024 * 1024),
    )(cmat, smat, q, k, seg)


def _corr_topk_kernel(pre_ref, pim_ref, c2_ref, s2_ref, attn_ref, delay_ref):
    corr = (_dot(pre_ref[...], c2_ref[...], ((1,), (0,))) -
            _dot(pim_ref[...], s2_ref[...], ((1,), (0,))))
    col = lax.broadcasted_iota(jnp.int32, (_BH, _L), 1)
    vals = []
    idxs = []
    cur = corr
    for _ in range(_K):
        m = jnp.max(cur, axis=1, keepdims=True)
        im = jnp.argmax(cur, axis=1, keepdims=True).astype(jnp.int32)
        vals.append(m)
        idxs.append(im)
        cur = jnp.where(col == im, -jnp.inf, cur)
    wts = jnp.concatenate(vals, axis=1)
    dly = jnp.concatenate(idxs, axis=1)
    mw = jnp.max(wts, axis=1, keepdims=True)
    e = jnp.exp(wts - mw)
    attn = e / jnp.sum(e, axis=1, keepdims=True)
    attn_ref[...] = jnp.concatenate(
        [attn, jnp.zeros((_BH, _KP - _K), jnp.float32)], axis=1)
    delay_ref[...] = jnp.concatenate(
        [dly, jnp.zeros((_BH, _KP - _K), jnp.int32)], axis=1)


def _corr_topk(pre_t, pim_t, c2mat, s2mat):
    return pl.pallas_call(
        _corr_topk_kernel,
        in_specs=[pl.BlockSpec((_BH, _FP), lambda: (0, 0)),
                  pl.BlockSpec((_BH, _FP), lambda: (0, 0)),
                  pl.BlockSpec((_FP, _L), lambda: (0, 0)),
                  pl.BlockSpec((_FP, _L), lambda: (0, 0))],
        out_specs=[pl.BlockSpec((_BH, _KP), lambda: (0, 0)),
                   pl.BlockSpec((_BH, _KP), lambda: (0, 0))],
        out_shape=[jax.ShapeDtypeStruct((_BH, _KP), jnp.float32),
                   jax.ShapeDtypeStruct((_BH, _KP), jnp.int32)],
    )(pre_t, pim_t, c2mat, s2mat)


def _agg_kernel(delay_ref, attn_ref, v_ref, ctx_ref, vv_ref):
    b = pl.program_id(0)
    for h in range(_HW):
        c0 = h * _DH
        vv_ref[0:_L, :] = v_ref[0, :, c0:c0 + _DH]
        vv_ref[_L:2 * _L, :] = v_ref[0, :, c0:c0 + _DH]
        g = b * _HW + h
        acc = (attn_ref[g, 0] *
               vv_ref[pl.ds(_L - delay_ref[g, 0], _L), :])
        for j in range(1, _K):
            d = delay_ref[g, j]
            acc = acc + (attn_ref[g, j] *
                         vv_ref[pl.ds(_L - d, _L), :])
        ctx_ref[0, :, c0:c0 + _DH] = acc


def _aggregate(delays, attn, v):
    return pl.pallas_call(
        _agg_kernel,
        grid=(_B,),
        in_specs=[pl.BlockSpec(memory_space=pltpu.SMEM),
                  pl.BlockSpec(memory_space=pltpu.SMEM),
                  pl.BlockSpec((1, _L, _DM), lambda b: (b, 0, 0))],
        out_specs=pl.BlockSpec((1, _L, _DM), lambda b: (b, 0, 0)),
        out_shape=jax.ShapeDtypeStruct((_B, _L, _DM), jnp.float32),
        scratch_shapes=[pltpu.VMEM((2 * _L, _DH), jnp.float32)],
    )(delays, attn, v)


def _out_kernel(x_ref, w_ref, b_ref, y_ref):
    y_ref[0] = _dot(x_ref[0], w_ref[...], ((1,), (1,)),
                    lax.Precision.DEFAULT) + b_ref[...]


def _out_proj(ctx, Wo, bo):
    nl = _L // _LB
    return pl.pallas_call(
        _out_kernel,
        grid=(_B, nl),
        in_specs=[pl.BlockSpec((1, _LB, _DM), lambda b, l: (b, l, 0)),
                  pl.BlockSpec((_DM, _DM), lambda b, l: (0, 0)),
                  pl.BlockSpec((1, _DM), lambda b, l: (0, 0))],
        out_specs=pl.BlockSpec((1, _LB, _DM), lambda b, l: (b, l, 0)),
        out_shape=jax.ShapeDtypeStruct((_B, _L, _DM), jnp.float32),
    )(ctx, Wo, bo.reshape(1, _DM))


def _aggregate_project(delays, attn, v, Wo, bo):
    return _out_proj(_aggregate(delays, attn, v), Wo, bo)


def kernel(x_q, x_kv, Wq, bq, Wk, bk, Wv, bv, Wo, bo):
    cmat = jnp.asarray(_C_NP)
    smat = jnp.asarray(_S_NP)
    c2mat = jnp.asarray(_C2_NP)
    s2mat = jnp.asarray(_S2_NP)
    seg = jnp.asarray(_SEG_NP)

    q, k, v = _projections(x_q, x_kv, Wq, bq, Wk, bk, Wv, bv)

    pre3, pim3 = _cross_spectrum(q, k, cmat, smat, seg)
    pre_t = pre3.transpose(0, 2, 1).reshape(_BH, _FP)
    pim_t = pim3.transpose(0, 2, 1).reshape(_BH, _FP)

    attn, delays = _corr_topk(pre_t, pim_t, c2mat, s2mat)

    return _aggregate_project(delays, attn, v, Wo, bo)
